# baseline (speedup 1.0000x reference)
import functools

import jax
import jax.numpy as jnp
from jax.experimental import pallas as pl
from jax.experimental.pallas import tpu as pltpu

H = 64
V = 64
LN_EPS = 1e-5
NORM_EPS = 1e-12

_LC = 512
_SUB = 32

_HP = jax.lax.Precision.HIGHEST


def _split_hi_lo(x):
    hi = x.astype(jnp.bfloat16)
    lo = (x - hi.astype(jnp.float32)).astype(jnp.bfloat16)
    return hi, lo


def _body(seq_ref, embed_ref, w1_ref, b1_ref, w2_ref, b2_ref, g_ref, bb_ref,
          wk_ref, wv_ref, wq_ref, wrp_ref, brp_ref, wout_ref, bout_ref,
          out_ref, tab_ref, z_ref, *, bb):
    j = pl.program_id(0)
    n_l = pl.num_programs(0)
    dn00 = (((0,), (0,)), ((), ()))
    dn01 = (((0,), (1,)), ((), ()))
    dn10 = (((1,), (0,)), ((), ()))

    @pl.when(j == 0)
    def _init():
        e = embed_ref[...]
        h1 = jnp.maximum(
            jnp.dot(e, w1_ref[...], precision=_HP,
                    preferred_element_type=jnp.float32) + b1_ref[...], 0.0)
        ff = jnp.dot(h1, w2_ref[...], precision=_HP,
                     preferred_element_type=jnp.float32) + b2_ref[...]
        x = e + ff
        mu = jnp.mean(x, axis=-1, keepdims=True)
        var = jnp.mean((x - mu) ** 2, axis=-1, keepdims=True)
        hs = (x - mu) * jax.lax.rsqrt(var + LN_EPS) * g_ref[...] + bb_ref[...]

        ktt = jax.lax.dot_general(wk_ref[...], hs, dn01, precision=_HP,
                                  preferred_element_type=jnp.float32)
        nrm = jnp.sqrt(jnp.sum(ktt * ktt, axis=0, keepdims=True))
        knt = ktt / jnp.maximum(nrm, NORM_EPS)
        vtt = jax.lax.dot_general(wv_ref[...], hs, dn01, precision=_HP,
                                  preferred_element_type=jnp.float32)
        wro = jnp.dot(wrp_ref[...], wout_ref[...], precision=_HP,
                      preferred_element_type=jnp.float32)
        vwt = jax.lax.dot_general(wro, vtt, dn00, precision=_HP,
                                  preferred_element_type=jnp.float32)

        kn_hi, kn_lo = _split_hi_lo(knt)
        tab_ref[...] = jnp.concatenate(
            [kn_hi, kn_lo, vwt.astype(jnp.bfloat16)], axis=0)

        qtt = jax.lax.dot_general(wq_ref[...], hs, dn01, precision=_HP,
                                  preferred_element_type=jnp.float32)
        q_hi, q_lo = _split_hi_lo(qtt)
        tokq = jnp.broadcast_to(
            seq_ref[0, :, (_LC - 1) * bb:_LC * bb], (V, bb))
        iota_v = jax.lax.broadcasted_iota(jnp.int32, (V, bb), 0)
        ohq = jnp.where(tokq == iota_v, 1.0, 0.0).astype(jnp.bfloat16)
        z_ref[...] = (
            jax.lax.dot_general(q_hi, ohq, dn10,
                                preferred_element_type=jnp.float32) +
            jax.lax.dot_general(q_lo, ohq, dn10,
                                preferred_element_type=jnp.float32))
        out_ref[...] = jnp.zeros(out_ref.shape, out_ref.dtype)

    mlast = jnp.where(j == 0, 0.0, 1.0)
    seq_row = seq_ref[0]
    tab4 = tab_ref[...]
    z = z_ref[...]
    acc = jnp.zeros((V, bb), jnp.float32)
    nsb = _SUB * bb
    iota_s = jax.lax.broadcasted_iota(jnp.int32, (V, nsb), 0)
    for ts in reversed(range(_LC // _SUB)):
        tok = jnp.broadcast_to(seq_row[:, ts * nsb:(ts + 1) * nsb], (V, nsb))
        oht = jnp.where(tok == iota_s, 1.0, 0.0).astype(jnp.bfloat16)
        res = jax.lax.dot_general(tab4, oht, dn10,
                                  preferred_element_type=jnp.float32)
        kts = res[0:H] + res[H:2 * H]
        wts = res[2 * H:3 * H]
        for lt in reversed(range(_SUB)):
            c0 = lt * bb
            kt = kts[:, c0:c0 + bb]
            vt = wts[:, c0:c0 + bb]
            m = kt * z
            m = m[0:32] + m[32:64]
            m = m[0:16] + m[16:32]
            m = m[0:8] + m[8:16]
            m = m + pltpu.roll(m, 4, axis=0)
            m = m + pltpu.roll(m, 2, axis=0)
            m = m + pltpu.roll(m, 1, axis=0)
            if ts * _SUB + lt == _LC - 1:
                m = m * mlast
            s = pltpu.repeat(m, 8, axis=0)
            acc = acc + s * vt
            z = z - s * kt
    z_ref[...] = z
    out_ref[...] = out_ref[...] + acc

    @pl.when(j == n_l - 1)
    def _fin():
        bro = jax.lax.dot_general(wout_ref[...], brp_ref[...], dn01,
                                  precision=_HP,
                                  preferred_element_type=jnp.float32)
        bro = bro + bout_ref[...]
        out_ref[...] = out_ref[...] + jnp.broadcast_to(bro, out_ref.shape)


def kernel(seq, embed, w1, b1, w2, b2, ln_g, ln_b, wk, wv, wq, wrp, brp,
           wout, bout):
    B, L = seq.shape
    n_l = L // _LC
    seq_r = seq.T.reshape(1, 1, L * B)
    row = lambda a: a.reshape(1, -1)

    full = lambda shape: pl.BlockSpec(shape, lambda j: (0, 0))
    out = pl.pallas_call(
        functools.partial(_body, bb=B),
        grid=(n_l,),
        in_specs=[
            pl.BlockSpec((1, 1, _LC * B),
                         lambda j, n_l=n_l: (0, 0, n_l - 1 - j)),
            full((V, H)),
            full((H, 2 * H)),
            full((1, 2 * H)),
            full((2 * H, H)),
            full((1, H)),
            full((1, H)),
            full((1, H)),
            full((H, H)),
            full((H, H)),
            full((H, H)),
            full((H, H)),
            full((1, H)),
            full((H, V)),
            pl.BlockSpec((V, 1), lambda j: (0, 0)),
        ],
        out_specs=pl.BlockSpec((V, B), lambda j: (0, 0)),
        out_shape=jax.ShapeDtypeStruct((V, B), jnp.float32),
        scratch_shapes=[
            pltpu.VMEM((3 * H, V), jnp.bfloat16),
            pltpu.VMEM((H, B), jnp.float32),
        ],
        compiler_params=pltpu.CompilerParams(
            dimension_semantics=("arbitrary",),
        ),
        name="delta_model",
    )(seq_r, embed, w1, row(b1), w2, row(b2), row(ln_g), row(ln_b),
      wk, wv, wq, wrp, row(brp), wout, bout.reshape(-1, 1))
    return out.T


# speedup vs baseline: 1.1952x; 1.1952x over previous
import functools

import jax
import jax.numpy as jnp
from jax.experimental import pallas as pl
from jax.experimental.pallas import tpu as pltpu

H = 64
V = 64
LN_EPS = 1e-5
NORM_EPS = 1e-12

_LC = 256
_SUB = 16

_HP = jax.lax.Precision.HIGHEST


def _split_hi_lo(x):
    hi = x.astype(jnp.bfloat16)
    lo = (x - hi.astype(jnp.float32)).astype(jnp.bfloat16)
    return hi, lo


def _body(seq_ref, embed_ref, w1_ref, b1_ref, w2_ref, b2_ref, g_ref, bb_ref,
          wk_ref, wv_ref, wq_ref, wrp_ref, brp_ref, wout_ref, bout_ref,
          out_ref, tab_ref, z_ref, *, bb):
    j = pl.program_id(0)
    n_l = pl.num_programs(0)
    dn00 = (((0,), (0,)), ((), ()))
    dn01 = (((0,), (1,)), ((), ()))
    dn10 = (((1,), (0,)), ((), ()))

    @pl.when(j == 0)
    def _init():
        e = embed_ref[...]
        h1 = jnp.maximum(
            jnp.dot(e, w1_ref[...], precision=_HP,
                    preferred_element_type=jnp.float32) + b1_ref[...], 0.0)
        ff = jnp.dot(h1, w2_ref[...], precision=_HP,
                     preferred_element_type=jnp.float32) + b2_ref[...]
        x = e + ff
        mu = jnp.mean(x, axis=-1, keepdims=True)
        var = jnp.mean((x - mu) ** 2, axis=-1, keepdims=True)
        hs = (x - mu) * jax.lax.rsqrt(var + LN_EPS) * g_ref[...] + bb_ref[...]

        ktt = jax.lax.dot_general(wk_ref[...], hs, dn01, precision=_HP,
                                  preferred_element_type=jnp.float32)
        nrm = jnp.sqrt(jnp.sum(ktt * ktt, axis=0, keepdims=True))
        knt = ktt / jnp.maximum(nrm, NORM_EPS)
        vtt = jax.lax.dot_general(wv_ref[...], hs, dn01, precision=_HP,
                                  preferred_element_type=jnp.float32)
        wro = jnp.dot(wrp_ref[...], wout_ref[...], precision=_HP,
                      preferred_element_type=jnp.float32)
        vwt = jax.lax.dot_general(wro, vtt, dn00, precision=_HP,
                                  preferred_element_type=jnp.float32)

        kn_hi, kn_lo = _split_hi_lo(knt)
        tab_ref[...] = jnp.concatenate([
            jnp.concatenate([kn_hi, kn_lo], axis=1),
            jnp.concatenate([vwt.astype(jnp.bfloat16),
                             jnp.zeros((H, V), jnp.bfloat16)], axis=1),
        ], axis=0)

        qtt = jax.lax.dot_general(wq_ref[...], hs, dn01, precision=_HP,
                                  preferred_element_type=jnp.float32)
        q_hi, q_lo = _split_hi_lo(qtt)
        tokq = jnp.broadcast_to(
            seq_ref[0, :, (_LC - 1) * bb:_LC * bb], (V, bb))
        iota_v = jax.lax.broadcasted_iota(jnp.int32, (V, bb), 0)
        ohq = jnp.where(tokq == iota_v, 1.0, 0.0).astype(jnp.bfloat16)
        z_ref[...] = (
            jax.lax.dot_general(q_hi, ohq, dn10,
                                preferred_element_type=jnp.float32) +
            jax.lax.dot_general(q_lo, ohq, dn10,
                                preferred_element_type=jnp.float32))
        out_ref[...] = jnp.zeros(out_ref.shape, out_ref.dtype)

    mlast = jnp.where(j == 0, 0.0, 1.0)
    seq_row = seq_ref[0]
    tab4 = tab_ref[...]
    z = z_ref[...]
    acc = jnp.zeros((V, bb), jnp.float32)
    nsb = _SUB * bb
    iota_s = jax.lax.broadcasted_iota(jnp.int32, (V, nsb), 0)
    for ts in reversed(range(_LC // _SUB)):
        tok = jnp.broadcast_to(seq_row[:, ts * nsb:(ts + 1) * nsb], (V, nsb))
        oht = jnp.where(tok == iota_s, 1.0, 0.0).astype(jnp.bfloat16)
        oh2 = pltpu.repeat(oht, 2, axis=0)
        res = jax.lax.dot_general(tab4, oh2, dn10,
                                  preferred_element_type=jnp.float32)
        kts = res[0:H]
        wts = res[H:2 * H]
        for lt in reversed(range(_SUB)):
            c0 = lt * bb
            kt = kts[:, c0:c0 + bb]
            vt = wts[:, c0:c0 + bb]
            m = kt * z
            m = m[0:32] + m[32:64]
            m = m[0:16] + m[16:32]
            m = m[0:8] + m[8:16]
            m = m + pltpu.roll(m, 4, axis=0)
            m = m + pltpu.roll(m, 2, axis=0)
            m = m + pltpu.roll(m, 1, axis=0)
            if ts * _SUB + lt == _LC - 1:
                m = m * mlast
            s = pltpu.repeat(m, 8, axis=0)
            acc = acc + s * vt
            z = z - s * kt
    z_ref[...] = z
    out_ref[...] = out_ref[...] + acc

    @pl.when(j == n_l - 1)
    def _fin():
        bro = jax.lax.dot_general(wout_ref[...], brp_ref[...], dn01,
                                  precision=_HP,
                                  preferred_element_type=jnp.float32)
        bro = bro + bout_ref[...]
        out_ref[...] = out_ref[...] + jnp.broadcast_to(bro, out_ref.shape)


def kernel(seq, embed, w1, b1, w2, b2, ln_g, ln_b, wk, wv, wq, wrp, brp,
           wout, bout):
    B, L = seq.shape
    n_l = L // _LC
    seq_r = seq.T.reshape(1, 1, L * B)
    row = lambda a: a.reshape(1, -1)

    full = lambda shape: pl.BlockSpec(shape, lambda j: (0, 0))
    out = pl.pallas_call(
        functools.partial(_body, bb=B),
        grid=(n_l,),
        in_specs=[
            pl.BlockSpec((1, 1, _LC * B),
                         lambda j, n_l=n_l: (0, 0, n_l - 1 - j)),
            full((V, H)),
            full((H, 2 * H)),
            full((1, 2 * H)),
            full((2 * H, H)),
            full((1, H)),
            full((1, H)),
            full((1, H)),
            full((H, H)),
            full((H, H)),
            full((H, H)),
            full((H, H)),
            full((1, H)),
            full((H, V)),
            pl.BlockSpec((V, 1), lambda j: (0, 0)),
        ],
        out_specs=pl.BlockSpec((V, B), lambda j: (0, 0)),
        out_shape=jax.ShapeDtypeStruct((V, B), jnp.float32),
        scratch_shapes=[
            pltpu.VMEM((2 * H, 2 * V), jnp.bfloat16),
            pltpu.VMEM((H, B), jnp.float32),
        ],
        compiler_params=pltpu.CompilerParams(
            dimension_semantics=("arbitrary",),
        ),
        name="delta_model",
    )(seq_r, embed, w1, row(b1), w2, row(b2), row(ln_g), row(ln_b),
      wk, wv, wq, wrp, row(brp), wout, bout.reshape(-1, 1))
    return out.T
